# Initial kernel scaffold; baseline (speedup 1.0000x reference)
import functools

import jax
import jax.numpy as jnp
import numpy as np
from jax.experimental import pallas as pl

D = 768
H = 12
DH = 64
WIN = 16
DEPTH = 2
MDIM = 64
T = 1.0
L = 2048

BLK = 128


def _ln(x, g, b):
    m = jnp.mean(x, axis=-1, keepdims=True)
    v = jnp.mean((x - m) ** 2, axis=-1, keepdims=True)
    return (x - m) / jnp.sqrt(v + 1e-6) * g + b


def _layer_body(x_ref, g1_ref, b1_ref, wqkv_ref, bqkv_ref, wp_ref, bp_ref,
                g2_ref, b2_ref, w1_ref, b1f_ref, w2_ref, b2f_ref, o_ref):
    x = x_ref[0]
    h = _ln(x, g1_ref[0], b1_ref[0])
    qkv = jnp.dot(h, wqkv_ref[...], preferred_element_type=jnp.float32) + bqkv_ref[0]
    wi = jax.lax.broadcasted_iota(jnp.int32, (BLK, BLK), 0) // WIN
    wj = jax.lax.broadcasted_iota(jnp.int32, (BLK, BLK), 1) // WIN
    mask = wi == wj
    scale = 1.0 / np.sqrt(DH)
    o = jnp.zeros((BLK, D), jnp.float32)
    for hh in range(H):
        qh = qkv[:, hh * DH:(hh + 1) * DH]
        kh = qkv[:, D + hh * DH:D + (hh + 1) * DH]
        vh = qkv[:, 2 * D + hh * DH:2 * D + (hh + 1) * DH]
        s = jnp.dot(qh, kh.T, preferred_element_type=jnp.float32) * scale
        s = jnp.where(mask, s, -jnp.inf)
        s = s - jnp.max(s, axis=-1, keepdims=True)
        p = jnp.exp(s)
        p = p / jnp.sum(p, axis=-1, keepdims=True)
        oh = jnp.dot(p, vh, preferred_element_type=jnp.float32)
        o = o.at[:, hh * DH:(hh + 1) * DH].set(oh)
    x = x + jnp.dot(o, wp_ref[...], preferred_element_type=jnp.float32) + bp_ref[0]
    h2 = _ln(x, g2_ref[0], b2_ref[0])
    m = jax.nn.gelu(jnp.dot(h2, w1_ref[...], preferred_element_type=jnp.float32) + b1f_ref[0])
    m = jnp.dot(m, w2_ref[...], preferred_element_type=jnp.float32) + b2f_ref[0]
    o_ref[0] = x + m


def _run_layer(x, g1, b1, wqkv, bqkv, wp, bp, g2, b2, w1, b1f, w2, b2f):
    Bb, Ll, _ = x.shape
    grid = (Bb, Ll // BLK)

    def full(shape):
        return pl.BlockSpec(shape, lambda bi, si: (0,) * len(shape))

    return pl.pallas_call(
        _layer_body,
        grid=grid,
        in_specs=[
            pl.BlockSpec((1, BLK, D), lambda bi, si: (bi, si, 0)),
            full((1, D)), full((1, D)),
            full((D, 3 * D)), full((1, 3 * D)),
            full((D, D)), full((1, D)),
            full((1, D)), full((1, D)),
            full((D, 4 * D)), full((1, 4 * D)),
            full((4 * D, D)), full((1, D)),
        ],
        out_specs=pl.BlockSpec((1, BLK, D), lambda bi, si: (bi, si, 0)),
        out_shape=jax.ShapeDtypeStruct((Bb, Ll, D), jnp.float32),
    )(x, g1[None], b1[None], wqkv, bqkv[None], wp, bp[None],
      g2[None], b2[None], w1, b1f[None], w2, b2f[None])


def _select(metric, r):
    n = metric.shape[0]
    e = n // 2
    mn = metric / (jnp.linalg.norm(metric, axis=-1, keepdims=True) + 1e-6)
    a, b = mn[::2], mn[1::2]
    sc = a @ b.T
    nmax = jnp.max(sc, axis=-1)
    nidx = jnp.argmax(sc, axis=-1).astype(jnp.int32)
    order = jnp.argsort(-nmax).astype(jnp.int32)
    inv = jnp.zeros((e,), jnp.int32).at[order].set(jnp.arange(e, dtype=jnp.int32))
    sel = inv < r
    unm = e - r
    fr_e = jnp.where(sel, unm + nidx, inv - r)
    wt_e = jnp.where(sel, jax.nn.sigmoid(nmax / T), 1.0)
    fr = jnp.stack([fr_e, unm + jnp.arange(e, dtype=jnp.int32)], axis=1).reshape(-1)
    wt = jnp.stack([wt_e, jnp.ones((e,), metric.dtype)], axis=1).reshape(-1)
    return fr, wt


def _merge(x1, x2, Wm, bm):
    r = 512
    m1 = x1 @ Wm[0] + bm[0]
    fr1, wt1 = _select(m1, r)
    s1 = jnp.zeros((1536,), x2.dtype).at[fr1].add(wt1)
    xw1 = jnp.zeros((1536, D), x2.dtype).at[fr1].add(wt1[:, None] * x2)
    xm1 = xw1 / jnp.maximum(s1[:, None], 1e-6)
    m2 = xm1 @ Wm[1] + bm[1]
    fr2, c2 = _select(m2, r)
    fr = fr2[fr1]
    wt = wt1 * c2[fr1]
    size = jnp.zeros((1024,), x2.dtype).at[fr].add(wt)
    xw2 = jnp.zeros((1024, D), x2.dtype).at[fr2].add(c2[:, None] * xw1)
    x_out = xw2 / jnp.maximum(size[:, None], 1e-6)
    src_out = jnp.zeros((1024, L), x2.dtype).at[fr, jnp.arange(L)].set(wt)
    return x_out, size[:, None], src_out


def kernel(hidden_states, ln1_g, ln1_b, Wqkv, bqkv, Wproj, bproj, ln2_g, ln2_b,
           Wfc1, bfc1, Wfc2, bfc2, Wm, bm):
    x = hidden_states
    xs = []
    for i in range(DEPTH):
        x = _run_layer(x, ln1_g[i], ln1_b[i], Wqkv[i], bqkv[i], Wproj[i],
                       bproj[i], ln2_g[i], ln2_b[i], Wfc1[i], bfc1[i],
                       Wfc2[i], bfc2[i])
        xs.append(x)
    x1, x2 = xs
    return jax.vmap(functools.partial(_merge, Wm=Wm, bm=bm))(x1, x2)


# speedup vs baseline: 1.7994x; 1.7994x over previous
import functools

import jax
import jax.numpy as jnp
import numpy as np
from jax.experimental import pallas as pl

D = 768
H = 12
DH = 64
WIN = 16
DEPTH = 2
MDIM = 64
T = 1.0
L = 2048
R_MERGE = 512


def _layer_norm(x, g, b):
    m = jnp.mean(x, axis=-1, keepdims=True)
    v = jnp.mean((x - m) ** 2, axis=-1, keepdims=True)
    return (x - m) / jnp.sqrt(v + 1e-6) * g + b


def _local_block(x, i, ln1_g, ln1_b, Wqkv, bqkv, Wproj, bproj, ln2_g, ln2_b,
                 Wfc1, bfc1, Wfc2, bfc2):
    Bb, Ll, d = x.shape
    h = _layer_norm(x, ln1_g[i], ln1_b[i])
    qkv = h @ Wqkv[i] + bqkv[i]
    nw = Ll // WIN
    qkv = qkv.reshape(Bb, nw, WIN, 3, H, DH)
    q = jnp.transpose(qkv[:, :, :, 0], (0, 1, 3, 2, 4))
    k = jnp.transpose(qkv[:, :, :, 1], (0, 1, 3, 2, 4))
    v = jnp.transpose(qkv[:, :, :, 2], (0, 1, 3, 2, 4))
    attn = jax.nn.softmax(q @ jnp.swapaxes(k, -1, -2) / np.sqrt(DH), axis=-1)
    o = jnp.transpose(attn @ v, (0, 1, 3, 2, 4)).reshape(Bb, Ll, d)
    x = x + o @ Wproj[i] + bproj[i]
    h2 = _layer_norm(x, ln2_g[i], ln2_b[i])
    m = jax.nn.gelu(h2 @ Wfc1[i] + bfc1[i]) @ Wfc2[i] + bfc2[i]
    return x + m


def _scores_body(a_ref, b_ref, nmax_ref, nidx_ref):
    a = a_ref[0]
    b = b_ref[0]
    sc = jnp.dot(a, b.T, preferred_element_type=jnp.float32)
    nmax_ref[0] = jnp.max(sc, axis=-1, keepdims=True)
    nidx_ref[0] = jnp.argmax(sc, axis=-1, keepdims=True).astype(jnp.int32)


def _node_scores(a, b):
    Bb, E, _ = a.shape
    nmax, nidx = pl.pallas_call(
        _scores_body,
        grid=(Bb,),
        in_specs=[pl.BlockSpec((1, E, MDIM), lambda i: (i, 0, 0)),
                  pl.BlockSpec((1, E, MDIM), lambda i: (i, 0, 0))],
        out_specs=(pl.BlockSpec((1, E, 1), lambda i: (i, 0, 0)),
                   pl.BlockSpec((1, E, 1), lambda i: (i, 0, 0))),
        out_shape=(jax.ShapeDtypeStruct((Bb, E, 1), jnp.float32),
                   jax.ShapeDtypeStruct((Bb, E, 1), jnp.int32)),
    )(a, b)
    return nmax[..., 0], nidx[..., 0]


def _mask_body(fr_ref, wt_ref, o_ref, *, rb):
    base = pl.program_id(1) * rb
    fr = fr_ref[0]
    wt = wt_ref[0]
    rows = base + jax.lax.broadcasted_iota(jnp.int32, (rb, fr.shape[-1]), 0)
    o_ref[0] = jnp.where(fr == rows, wt, 0.0)


def _build_source(fr, wt, nrows, rb=512):
    Bb, Ltot = fr.shape
    return pl.pallas_call(
        functools.partial(_mask_body, rb=rb),
        grid=(Bb, nrows // rb),
        in_specs=[pl.BlockSpec((1, 1, Ltot), lambda i, j: (i, 0, 0)),
                  pl.BlockSpec((1, 1, Ltot), lambda i, j: (i, 0, 0))],
        out_specs=pl.BlockSpec((1, rb, Ltot), lambda i, j: (i, j, 0)),
        out_shape=jax.ShapeDtypeStruct((Bb, nrows, Ltot), jnp.float32),
    )(fr[:, None, :].astype(jnp.int32), wt[:, None, :])


def _final_body(fr_ref, wt_ref, x2_ref, src_ref, size_ref, x_ref, *, rb):
    base = pl.program_id(1) * rb
    fr = fr_ref[0]
    wt = wt_ref[0]
    rows = base + jax.lax.broadcasted_iota(jnp.int32, (rb, L), 0)
    s_blk = jnp.where(fr == rows, wt, 0.0)
    src_ref[0] = s_blk
    size = jnp.sum(s_blk, axis=-1, keepdims=True)
    size_ref[0] = size
    xw = jnp.dot(s_blk, x2_ref[0], preferred_element_type=jnp.float32)
    x_ref[0] = xw / jnp.maximum(size, 1e-6)


def _final_combine(fr, wt, x2, nrows, rb=512):
    Bb = fr.shape[0]
    return pl.pallas_call(
        functools.partial(_final_body, rb=rb),
        grid=(Bb, nrows // rb),
        in_specs=[pl.BlockSpec((1, 1, L), lambda i, j: (i, 0, 0)),
                  pl.BlockSpec((1, 1, L), lambda i, j: (i, 0, 0)),
                  pl.BlockSpec((1, L, D), lambda i, j: (i, 0, 0))],
        out_specs=(pl.BlockSpec((1, rb, L), lambda i, j: (i, j, 0)),
                   pl.BlockSpec((1, rb, 1), lambda i, j: (i, j, 0)),
                   pl.BlockSpec((1, rb, D), lambda i, j: (i, j, 0))),
        out_shape=(jax.ShapeDtypeStruct((Bb, nrows, L), jnp.float32),
                   jax.ShapeDtypeStruct((Bb, nrows, 1), jnp.float32),
                   jax.ShapeDtypeStruct((Bb, nrows, D), jnp.float32)),
    )(fr[:, None, :].astype(jnp.int32), wt[:, None, :], x2)


def _selection(nmax, nidx):
    E = nmax.shape[0]
    unm = E - R_MERGE
    order = jnp.argsort(-nmax).astype(jnp.int32)
    inv = jnp.zeros((E,), jnp.int32).at[order].set(jnp.arange(E, dtype=jnp.int32))
    sel = inv < R_MERGE
    fr_e = jnp.where(sel, unm + nidx, inv - R_MERGE)
    wt_e = jnp.where(sel, jax.nn.sigmoid(nmax / T), 1.0)
    fr = jnp.stack([fr_e, unm + jnp.arange(E, dtype=jnp.int32)], axis=1).reshape(-1)
    wt = jnp.stack([wt_e, jnp.ones((E,), nmax.dtype)], axis=1).reshape(-1)
    return fr, wt, order


def _sizes_step1(nmax, nidx, order):
    src_idx = order[:R_MERGE]
    w = jax.nn.sigmoid(jnp.take(nmax, src_idx) / T)
    dst = jnp.take(nidx, src_idx)
    merged = jnp.ones((L // 2,), nmax.dtype).at[dst].add(jnp.ones_like(w) * w)
    return jnp.concatenate([jnp.ones((L // 2 - R_MERGE,), nmax.dtype), merged])


def kernel(hidden_states, ln1_g, ln1_b, Wqkv, bqkv, Wproj, bproj, ln2_g, ln2_b,
           Wfc1, bfc1, Wfc2, bfc2, Wm, bm):
    x = hidden_states
    xs = []
    for i in range(DEPTH):
        x = _local_block(x, i, ln1_g, ln1_b, Wqkv, bqkv, Wproj, bproj,
                         ln2_g, ln2_b, Wfc1, bfc1, Wfc2, bfc2)
        xs.append(x)
    x1, x2 = xs

    m1 = x1 @ Wm[0] + bm[0]
    mn1 = m1 / (jnp.linalg.norm(m1, axis=-1, keepdims=True) + 1e-6)
    nmax1, nidx1 = _node_scores(mn1[:, ::2], mn1[:, 1::2])
    fr1, wt1, order1 = jax.vmap(_selection)(nmax1, nidx1)
    s1 = jax.vmap(_sizes_step1)(nmax1, nidx1, order1)

    M1 = _build_source(fr1, wt1, L - R_MERGE)
    xm1 = jnp.einsum("brl,bld->brd", M1, x2) / jnp.maximum(s1[..., None], 1e-6)

    m2 = xm1 @ Wm[1] + bm[1]
    mn2 = m2 / (jnp.linalg.norm(m2, axis=-1, keepdims=True) + 1e-6)
    nmax2, nidx2 = _node_scores(mn2[:, ::2], mn2[:, 1::2])
    fr2, c2, _ = jax.vmap(_selection)(nmax2, nidx2)

    fr = jnp.take_along_axis(fr2, fr1, axis=-1)
    wt = wt1 * jnp.take_along_axis(c2, fr1, axis=-1)
    src, size, x_out = _final_combine(fr, wt, x2, L - 2 * R_MERGE)
    return x_out, size, src
